# Initial kernel scaffold; baseline (speedup 1.0000x reference)
import functools

import jax
import jax.numpy as jnp
from jax import lax
from jax.experimental import pallas as pl
from jax.experimental.pallas import tpu as pltpu
from jax.experimental.pallas import tpu_sc as plsc

N = 10000
E = 320000
D = 128

NC = 2
NS = 16
NW = NC * NS

CHUNK = 128
EDGES_PER_W = 10240
E_PAD = EDGES_PER_W * NW
N_CHUNKS = EDGES_PER_W // CHUNK

ROWS_PER_TILE = N // NS


def _mm_kernel(x_ref, w_ref, b_ref, o_ref):
    o_ref[...] = jnp.dot(x_ref[...], w_ref[...],
                         preferred_element_type=jnp.float32) + b_ref[...]


def _mm_fused_kernel(p0_ref, p1_ref, w_ref, b_ref, o_ref):
    h = jnp.maximum(p0_ref[...] + p1_ref[...], 0.0)
    o_ref[...] = jnp.dot(h, w_ref[...],
                         preferred_element_type=jnp.float32) + b_ref[...]


def _add_relu_kernel(p0_ref, p1_ref, o_ref):
    o_ref[...] = jnp.maximum(p0_ref[...] + p1_ref[...], 0.0)


_BLK = 2000


def _linear(x, W, b):
    return pl.pallas_call(
        _mm_kernel,
        out_shape=jax.ShapeDtypeStruct((N, D), jnp.float32),
        grid=(N // _BLK,),
        in_specs=[
            pl.BlockSpec((_BLK, D), lambda i: (i, 0)),
            pl.BlockSpec((D, D), lambda i: (0, 0)),
            pl.BlockSpec((1, D), lambda i: (0, 0)),
        ],
        out_specs=pl.BlockSpec((_BLK, D), lambda i: (i, 0)),
    )(x, W, b.reshape(1, D))


def _linear_fused(p0, p1, W, b):
    return pl.pallas_call(
        _mm_fused_kernel,
        out_shape=jax.ShapeDtypeStruct((N, D), jnp.float32),
        grid=(N // _BLK,),
        in_specs=[
            pl.BlockSpec((_BLK, D), lambda i: (i, 0)),
            pl.BlockSpec((_BLK, D), lambda i: (i, 0)),
            pl.BlockSpec((D, D), lambda i: (0, 0)),
            pl.BlockSpec((1, D), lambda i: (0, 0)),
        ],
        out_specs=pl.BlockSpec((_BLK, D), lambda i: (i, 0)),
    )(p0, p1, W, b.reshape(1, D))


def _add_relu(p0, p1):
    return pl.pallas_call(
        _add_relu_kernel,
        out_shape=jax.ShapeDtypeStruct((N, D), jnp.float32),
        grid=(N // _BLK,),
        in_specs=[
            pl.BlockSpec((_BLK, D), lambda i: (i, 0)),
            pl.BlockSpec((_BLK, D), lambda i: (i, 0)),
        ],
        out_specs=pl.BlockSpec((_BLK, D), lambda i: (i, 0)),
    )(p0, p1)


_sc_mesh = plsc.VectorSubcoreMesh(core_axis_name="c", subcore_axis_name="s")


@functools.partial(
    pl.kernel,
    out_type=jax.ShapeDtypeStruct((NC, N, D), jnp.float32),
    mesh=_sc_mesh,
    scratch_types=[
        pltpu.VMEM((CHUNK,), jnp.int32),
        pltpu.VMEM((CHUNK,), jnp.int32),
        pltpu.VMEM((CHUNK,), jnp.float32),
        pltpu.VMEM((CHUNK, D), jnp.float32),
        pltpu.VMEM_SHARED((N, D), jnp.float32),
        pltpu.SemaphoreType.DMA,
    ],
)
def _sc_scatter(h_hbm, src_hbm, dst_hbm, w_hbm, out_hbm,
                src_v, dst_v, w_v, rows_v, acc, sem):
    cid = lax.axis_index("c")
    sid = lax.axis_index("s")
    wid = cid * NS + sid

    def _zero_body(i, _):
        rows_v[i // 8, pl.ds((i % 8) * 16, 16)] = jnp.zeros((16,), jnp.float32)
        return 0
    lax.fori_loop(0, CHUNK * 8, _zero_body, 0)

    row0 = sid * ROWS_PER_TILE
    n_full = ROWS_PER_TILE // CHUNK
    rem = ROWS_PER_TILE - n_full * CHUNK
    for k in range(n_full):
        pltpu.sync_copy(rows_v, acc.at[pl.ds(row0 + k * CHUNK, CHUNK)])
    pltpu.sync_copy(rows_v.at[pl.ds(0, rem)],
                    acc.at[pl.ds(row0 + n_full * CHUNK, rem)])

    plsc.subcore_barrier()

    def _edge_body(i, _):
        base = wid * EDGES_PER_W + i * CHUNK
        pltpu.sync_copy(src_hbm.at[pl.ds(base, CHUNK)], src_v)
        pltpu.sync_copy(dst_hbm.at[pl.ds(base, CHUNK)], dst_v)
        pltpu.sync_copy(w_hbm.at[pl.ds(base, CHUNK)], w_v)
        pltpu.async_copy(h_hbm.at[src_v], rows_v, sem).wait()

        def _scale_body(e, _):
            w = w_v[e]
            for j in range(D // 16):
                rows_v[e, pl.ds(j * 16, 16)] = rows_v[e, pl.ds(j * 16, 16)] * w
            return 0
        lax.fori_loop(0, CHUNK, _scale_body, 0)

        pltpu.sync_copy(rows_v, acc.at[dst_v], add=True)
        return 0
    lax.fori_loop(0, N_CHUNKS, _edge_body, 0)

    plsc.subcore_barrier()

    for k in range(n_full):
        pltpu.sync_copy(acc.at[pl.ds(row0 + k * CHUNK, CHUNK)], rows_v)
        pltpu.sync_copy(rows_v, out_hbm.at[cid, pl.ds(row0 + k * CHUNK, CHUNK)])
    pltpu.sync_copy(acc.at[pl.ds(row0 + n_full * CHUNK, rem)],
                    rows_v.at[pl.ds(0, rem)])
    pltpu.sync_copy(rows_v.at[pl.ds(0, rem)],
                    out_hbm.at[cid, pl.ds(row0 + n_full * CHUNK, rem)])


def kernel(feat, edge_index, edge_weight, W1, b1, W2, b2):
    src = edge_index[0]
    dst = edge_index[1]
    pad = E_PAD - E
    src_p = jnp.concatenate([src, jnp.zeros((pad,), jnp.int32)])
    dst_p = jnp.concatenate([dst, jnp.zeros((pad,), jnp.int32)])
    w_p = jnp.concatenate([edge_weight, jnp.zeros((pad,), jnp.float32)])

    h1 = _linear(feat, W1, b1)
    p1 = _sc_scatter(h1, src_p, dst_p, w_p)
    h2 = _linear_fused(p1[0], p1[1], W2, b2)
    p2 = _sc_scatter(h2, src_p, dst_p, w_p)
    return _add_relu(p2[0], p2[1])


# speedup vs baseline: 2.4975x; 2.4975x over previous
import functools

import jax
import jax.numpy as jnp
from jax import lax
from jax.experimental import pallas as pl
from jax.experimental.pallas import tpu as pltpu
from jax.experimental.pallas import tpu_sc as plsc

N = 10000
E = 320000
D = 128

NC = 2
NS = 16
NW = NC * NS

CHUNK = 128
EDGES_PER_W = 10240
E_PAD = EDGES_PER_W * NW
N_CHUNKS = EDGES_PER_W // CHUNK

ROWS_PER_TILE = 624
EXTRA_ROW0 = NS * ROWS_PER_TILE
EXTRA_ROWS = N - EXTRA_ROW0


def _mm_kernel(x_ref, w_ref, b_ref, o_ref):
    o_ref[...] = jnp.dot(x_ref[...], w_ref[...],
                         preferred_element_type=jnp.float32) + b_ref[...]


def _mm_fused_kernel(p0_ref, p1_ref, w_ref, b_ref, o_ref):
    h = jnp.maximum(p0_ref[...] + p1_ref[...], 0.0)
    o_ref[...] = jnp.dot(h, w_ref[...],
                         preferred_element_type=jnp.float32) + b_ref[...]


def _add_relu_kernel(p0_ref, p1_ref, o_ref):
    o_ref[...] = jnp.maximum(p0_ref[...] + p1_ref[...], 0.0)


_BLK = 2000


def _linear(x, W, b):
    return pl.pallas_call(
        _mm_kernel,
        out_shape=jax.ShapeDtypeStruct((N, D), jnp.float32),
        grid=(N // _BLK,),
        in_specs=[
            pl.BlockSpec((_BLK, D), lambda i: (i, 0)),
            pl.BlockSpec((D, D), lambda i: (0, 0)),
            pl.BlockSpec((1, D), lambda i: (0, 0)),
        ],
        out_specs=pl.BlockSpec((_BLK, D), lambda i: (i, 0)),
    )(x, W, b.reshape(1, D))


def _linear_fused(p0, p1, W, b):
    return pl.pallas_call(
        _mm_fused_kernel,
        out_shape=jax.ShapeDtypeStruct((N, D), jnp.float32),
        grid=(N // _BLK,),
        in_specs=[
            pl.BlockSpec((_BLK, D), lambda i: (i, 0)),
            pl.BlockSpec((_BLK, D), lambda i: (i, 0)),
            pl.BlockSpec((D, D), lambda i: (0, 0)),
            pl.BlockSpec((1, D), lambda i: (0, 0)),
        ],
        out_specs=pl.BlockSpec((_BLK, D), lambda i: (i, 0)),
    )(p0, p1, W, b.reshape(1, D))


def _add_relu(p0, p1):
    return pl.pallas_call(
        _add_relu_kernel,
        out_shape=jax.ShapeDtypeStruct((N, D), jnp.float32),
        grid=(N // _BLK,),
        in_specs=[
            pl.BlockSpec((_BLK, D), lambda i: (i, 0)),
            pl.BlockSpec((_BLK, D), lambda i: (i, 0)),
        ],
        out_specs=pl.BlockSpec((_BLK, D), lambda i: (i, 0)),
    )(p0, p1)


_sc_mesh = plsc.VectorSubcoreMesh(core_axis_name="c", subcore_axis_name="s")


@functools.partial(
    pl.kernel,
    out_type=jax.ShapeDtypeStruct((NC, N, D), jnp.float32),
    mesh=_sc_mesh,
    scratch_types=[
        pltpu.VMEM((CHUNK,), jnp.int32),
        pltpu.VMEM((CHUNK,), jnp.int32),
        pltpu.VMEM((CHUNK,), jnp.float32),
        pltpu.VMEM((CHUNK, D), jnp.float32),
        pltpu.VMEM_SHARED((N, D), jnp.float32),
        pltpu.SemaphoreType.DMA,
    ],
)
def _sc_scatter(h_hbm, src_hbm, dst_hbm, w_hbm, out_hbm,
                src_v, dst_v, w_v, rows_v, acc, sem):
    cid = lax.axis_index("c")
    sid = lax.axis_index("s")
    wid = cid * NS + sid

    def _zero_body(i, _):
        rows_v[i // 8, pl.ds((i % 8) * 16, 16)] = jnp.zeros((16,), jnp.float32)
        return 0
    lax.fori_loop(0, CHUNK * 8, _zero_body, 0)

    row0 = sid * ROWS_PER_TILE
    n_full = ROWS_PER_TILE // CHUNK
    rem = ROWS_PER_TILE - n_full * CHUNK
    for k in range(n_full):
        pltpu.sync_copy(rows_v, acc.at[pl.ds(row0 + k * CHUNK, CHUNK)])
    pltpu.sync_copy(rows_v.at[pl.ds(0, rem)],
                    acc.at[pl.ds(row0 + n_full * CHUNK, rem)])

    @pl.when(sid == NS - 1)
    def _zero_extra():
        pltpu.sync_copy(rows_v.at[pl.ds(0, EXTRA_ROWS)],
                        acc.at[pl.ds(EXTRA_ROW0, EXTRA_ROWS)])

    plsc.subcore_barrier()

    def _edge_body(i, _):
        base = wid * EDGES_PER_W + i * CHUNK
        pltpu.sync_copy(src_hbm.at[pl.ds(base, CHUNK)], src_v)
        pltpu.sync_copy(dst_hbm.at[pl.ds(base, CHUNK)], dst_v)
        pltpu.sync_copy(w_hbm.at[pl.ds(base, CHUNK)], w_v)
        pltpu.async_copy(h_hbm.at[src_v], rows_v, sem).wait()

        def _scale_body(g, _):
            wv = w_v[pl.ds(g * 16, 16)]
            for l in range(16):
                w = wv[l]
                e = g * 16 + l
                for j in range(D // 16):
                    rows_v[e, pl.ds(j * 16, 16)] = (
                        rows_v[e, pl.ds(j * 16, 16)] * w)
            return 0
        lax.fori_loop(0, CHUNK // 16, _scale_body, 0)

        pltpu.sync_copy(rows_v, acc.at[dst_v], add=True)
        return 0
    lax.fori_loop(0, N_CHUNKS, _edge_body, 0)

    plsc.subcore_barrier()

    for k in range(n_full):
        pltpu.sync_copy(acc.at[pl.ds(row0 + k * CHUNK, CHUNK)], rows_v)
        pltpu.sync_copy(rows_v, out_hbm.at[cid, pl.ds(row0 + k * CHUNK, CHUNK)])
    pltpu.sync_copy(acc.at[pl.ds(row0 + n_full * CHUNK, rem)],
                    rows_v.at[pl.ds(0, rem)])
    pltpu.sync_copy(rows_v.at[pl.ds(0, rem)],
                    out_hbm.at[cid, pl.ds(row0 + n_full * CHUNK, rem)])

    @pl.when(sid == NS - 1)
    def _copy_extra():
        pltpu.sync_copy(acc.at[pl.ds(EXTRA_ROW0, EXTRA_ROWS)],
                        rows_v.at[pl.ds(0, EXTRA_ROWS)])
        pltpu.sync_copy(rows_v.at[pl.ds(0, EXTRA_ROWS)],
                        out_hbm.at[cid, pl.ds(EXTRA_ROW0, EXTRA_ROWS)])


def kernel(feat, edge_index, edge_weight, W1, b1, W2, b2):
    src = edge_index[0]
    dst = edge_index[1]
    pad = E_PAD - E
    src_p = jnp.concatenate([src, jnp.zeros((pad,), jnp.int32)])
    dst_p = jnp.concatenate([dst, jnp.zeros((pad,), jnp.int32)])
    w_p = jnp.concatenate([edge_weight, jnp.zeros((pad,), jnp.float32)])

    h1 = _linear(feat, W1, b1)
    p1 = _sc_scatter(h1, src_p, dst_p, w_p)
    h2 = _linear_fused(p1[0], p1[1], W2, b2)
    p2 = _sc_scatter(h2, src_p, dst_p, w_p)
    return _add_relu(p2[0], p2[1])


# speedup vs baseline: 2.6740x; 1.0707x over previous
import functools

import jax
import jax.numpy as jnp
from jax import lax
from jax.experimental import pallas as pl
from jax.experimental.pallas import tpu as pltpu
from jax.experimental.pallas import tpu_sc as plsc

N = 10000
E = 320000
D = 128

NC = 2
NS = 16
NW = NC * NS

CHUNK = 32
_NBUF = 3
NHALF = 3
EDGES_PER_W = 10368
E_PAD = EDGES_PER_W * NW
N_CHUNKS = EDGES_PER_W // (CHUNK * NHALF)
LEN = N_CHUNKS * CHUNK

ROWS_PER_TILE = 624
EXTRA_ROW0 = NS * ROWS_PER_TILE
EXTRA_ROWS = N - EXTRA_ROW0
ZROWS = 128


def _mm_kernel(x_ref, w_ref, b_ref, o_ref):
    o_ref[...] = jnp.dot(x_ref[...], w_ref[...],
                         preferred_element_type=jnp.float32) + b_ref[...]


def _mm_fused_kernel(p0_ref, p1_ref, w_ref, b_ref, o_ref):
    h = jnp.maximum(p0_ref[...] + p1_ref[...], 0.0)
    o_ref[...] = jnp.dot(h, w_ref[...],
                         preferred_element_type=jnp.float32) + b_ref[...]


def _add_relu_kernel(p0_ref, p1_ref, o_ref):
    o_ref[...] = jnp.maximum(p0_ref[...] + p1_ref[...], 0.0)


_BLK = 2000


def _linear(x, W, b):
    return pl.pallas_call(
        _mm_kernel,
        out_shape=jax.ShapeDtypeStruct((N, D), jnp.float32),
        grid=(N // _BLK,),
        in_specs=[
            pl.BlockSpec((_BLK, D), lambda i: (i, 0)),
            pl.BlockSpec((D, D), lambda i: (0, 0)),
            pl.BlockSpec((1, D), lambda i: (0, 0)),
        ],
        out_specs=pl.BlockSpec((_BLK, D), lambda i: (i, 0)),
    )(x, W, b.reshape(1, D))


def _linear_fused(p0, p1, W, b):
    return pl.pallas_call(
        _mm_fused_kernel,
        out_shape=jax.ShapeDtypeStruct((N, D), jnp.float32),
        grid=(N // _BLK,),
        in_specs=[
            pl.BlockSpec((_BLK, D), lambda i: (i, 0)),
            pl.BlockSpec((_BLK, D), lambda i: (i, 0)),
            pl.BlockSpec((D, D), lambda i: (0, 0)),
            pl.BlockSpec((1, D), lambda i: (0, 0)),
        ],
        out_specs=pl.BlockSpec((_BLK, D), lambda i: (i, 0)),
    )(p0, p1, W, b.reshape(1, D))


def _add_relu(p0, p1):
    return pl.pallas_call(
        _add_relu_kernel,
        out_shape=jax.ShapeDtypeStruct((N, D), jnp.float32),
        grid=(N // _BLK,),
        in_specs=[
            pl.BlockSpec((_BLK, D), lambda i: (i, 0)),
            pl.BlockSpec((_BLK, D), lambda i: (i, 0)),
        ],
        out_specs=pl.BlockSpec((_BLK, D), lambda i: (i, 0)),
    )(p0, p1)


_sc_mesh = plsc.VectorSubcoreMesh(core_axis_name="c", subcore_axis_name="s")


@functools.partial(
    pl.kernel,
    out_type=jax.ShapeDtypeStruct((NC, N, D), jnp.float32),
    mesh=_sc_mesh,
    scratch_types=[
        pltpu.VMEM((LEN,), jnp.int32),
        pltpu.VMEM((LEN,), jnp.int32),
        pltpu.VMEM((LEN,), jnp.float32),
        [pltpu.VMEM((CHUNK, D), jnp.float32) for _ in range(_NBUF)],
        pltpu.VMEM_SHARED((N, D), jnp.float32),
        [pltpu.SemaphoreType.DMA for _ in range(_NBUF)],
        [pltpu.SemaphoreType.DMA for _ in range(_NBUF)],
        pltpu.SemaphoreType.DMA,
    ],
)
def _sc_scatter(h_hbm, src_hbm, dst_hbm, w_hbm, out_hbm,
                src_v, dst_v, w_v, rows, acc, gsems, ssems, zsem):
    cid = lax.axis_index("c")
    sid = lax.axis_index("s")
    wid = cid * NS + sid

    def _zero_body(i, _):
        rows[0][i // 8, pl.ds((i % 8) * 16, 16)] = (
            jnp.zeros((16,), jnp.float32))
        return 0
    lax.fori_loop(0, CHUNK * 8, _zero_body, 0)

    row0 = sid * ROWS_PER_TILE
    n_full = ROWS_PER_TILE // CHUNK
    rem = ROWS_PER_TILE - n_full * CHUNK

    def _zissue(k, _):
        pltpu.async_copy(rows[0], acc.at[pl.ds(row0 + k * CHUNK, CHUNK)],
                         zsem)
        return 0
    lax.fori_loop(0, n_full, _zissue, 0)
    pltpu.sync_copy(rows[0].at[pl.ds(0, rem)],
                    acc.at[pl.ds(row0 + n_full * CHUNK, rem)])

    @pl.when(sid == NS - 1)
    def _zero_extra():
        pltpu.sync_copy(rows[0].at[pl.ds(0, EXTRA_ROWS)],
                        acc.at[pl.ds(EXTRA_ROW0, EXTRA_ROWS)])

    def _zwait(k, _):
        pltpu.make_async_copy(rows[0], acc.at[pl.ds(row0, CHUNK)],
                              zsem).wait()
        return 0
    lax.fori_loop(0, n_full, _zwait, 0)

    plsc.subcore_barrier()

    def _gather(i, b):
        return pltpu.async_copy(h_hbm.at[src_v.at[pl.ds(i * CHUNK, CHUNK)]],
                                rows[b], gsems[b])

    def _scatter(i, b):
        return pltpu.async_copy(rows[b],
                                acc.at[dst_v.at[pl.ds(i * CHUNK, CHUNK)]],
                                ssems[b], add=True)

    def _wait_gather(i, b):
        pltpu.make_async_copy(h_hbm.at[src_v.at[pl.ds(i * CHUNK, CHUNK)]],
                              rows[b], gsems[b]).wait()

    def _wait_scatter(i, b):
        pltpu.make_async_copy(rows[b],
                              acc.at[dst_v.at[pl.ds(i * CHUNK, CHUNK)]],
                              ssems[b]).wait()

    def _scale(i, b):
        def _scale_body(g, _):
            wv = w_v[pl.ds(i * CHUNK + g * 16, 16)]
            for l in range(16):
                w = wv[l]
                e = g * 16 + l
                for j in range(D // 16):
                    rows[b][e, pl.ds(j * 16, 16)] = (
                        rows[b][e, pl.ds(j * 16, 16)] * w)
            return 0
        lax.fori_loop(0, CHUNK // 16, _scale_body, 0)

    def _half_body(h, _):
        base = wid * EDGES_PER_W + h * LEN
        pltpu.sync_copy(src_hbm.at[pl.ds(base, LEN)], src_v)
        pltpu.sync_copy(dst_hbm.at[pl.ds(base, LEN)], dst_v)
        pltpu.sync_copy(w_hbm.at[pl.ds(base, LEN)], w_v)

        for b in range(_NBUF - 1):
            _gather(b, b)

        def _super_body(s, _):
            for b in range(_NBUF):
                i = s * _NBUF + b
                _wait_gather(i, b)
                _scale(i, b)
                _scatter(i, b)
                nxt = i + _NBUF - 1
                nb = (b + _NBUF - 1) % _NBUF

                @pl.when(jnp.logical_and(nxt < N_CHUNKS, i >= 1))
                def _drain_prev(i=i, nb=nb):
                    _wait_scatter(i - 1, nb)

                @pl.when(nxt < N_CHUNKS)
                def _launch_next(nxt=nxt, nb=nb):
                    _gather(nxt, nb)
            return 0
        lax.fori_loop(0, N_CHUNKS // _NBUF, _super_body, 0)

        for k in range(_NBUF):
            i_last = N_CHUNKS - _NBUF + k
            _wait_scatter(i_last, i_last % _NBUF)
        return 0
    lax.fori_loop(0, NHALF, _half_body, 0)

    plsc.subcore_barrier()

    def _cp_body(k, _):
        pltpu.sync_copy(acc.at[pl.ds(row0 + k * CHUNK, CHUNK)], rows[0])
        pltpu.sync_copy(rows[0],
                        out_hbm.at[cid, pl.ds(row0 + k * CHUNK, CHUNK)])
        return 0
    lax.fori_loop(0, n_full, _cp_body, 0)
    pltpu.sync_copy(acc.at[pl.ds(row0 + n_full * CHUNK, rem)],
                    rows[0].at[pl.ds(0, rem)])
    pltpu.sync_copy(rows[0].at[pl.ds(0, rem)],
                    out_hbm.at[cid, pl.ds(row0 + n_full * CHUNK, rem)])

    @pl.when(sid == NS - 1)
    def _copy_extra():
        pltpu.sync_copy(acc.at[pl.ds(EXTRA_ROW0, EXTRA_ROWS)],
                        rows[1].at[pl.ds(0, EXTRA_ROWS)])
        pltpu.sync_copy(rows[1].at[pl.ds(0, EXTRA_ROWS)],
                        out_hbm.at[cid, pl.ds(EXTRA_ROW0, EXTRA_ROWS)])


def kernel(feat, edge_index, edge_weight, W1, b1, W2, b2):
    src = edge_index[0]
    dst = edge_index[1]
    pad = E_PAD - E
    src_p = jnp.concatenate([src, jnp.zeros((pad,), jnp.int32)])
    dst_p = jnp.concatenate([dst, jnp.zeros((pad,), jnp.int32)])
    w_p = jnp.concatenate([edge_weight, jnp.zeros((pad,), jnp.float32)])

    h1 = _linear(feat, W1, b1)
    p1 = _sc_scatter(h1, src_p, dst_p, w_p)
    h2 = _linear_fused(p1[0], p1[1], W2, b2)
    p2 = _sc_scatter(h2, src_p, dst_p, w_p)
    return _add_relu(p2[0], p2[1])
